# baseline (speedup 1.0000x reference)
import functools

import jax
import jax.numpy as jnp
import numpy as np
from jax import lax
from jax.experimental import pallas as pl
from jax.experimental.pallas import tpu as pltpu
from jax.experimental.pallas import tpu_sc as plsc

N = 10000
E = 320000
D = 128
H = 128
NC_OUT = 10

NPAD = 10240
NTILES = 32
CHUNK = 128
BLKCH = 16
NBLK = 5
NCH = NBLK * BLKCH
EPT = NCH * CHUNK
EPAD = EPT * NTILES
ROWS_PT = NPAD // 16

_mesh = plsc.VectorSubcoreMesh(core_axis_name="c", subcore_axis_name="s")


NSUB = 4


def _sc_agg_body(h_hbm, src_hbm, dst_hbm, alive_hbm, aggp_hbm, cntp_hbm,
                 sidx, didx, agg_sh,
                 g00, g01, g02, g03, g10, g11, g12, g13, ssem0, ssem1):
    c = lax.axis_index("c")
    s = lax.axis_index("s")
    wid = c * 16 + s

    def phase_cnt(alive_v, cnt_v):
        pltpu.sync_copy(alive_hbm, alive_v)

        @pl.loop(0, NPAD // 16)
        def _(i):
            cnt_v[pl.ds(i * 16, 16)] = jnp.zeros((16,), jnp.float32)

        @pl.loop(0, NBLK)
        def _(b):
            pltpu.sync_copy(src_hbm.at[wid, b], sidx)
            pltpu.sync_copy(dst_hbm.at[wid, b], didx)

            @pl.loop(0, BLKCH * 8)
            def _(t):
                j = t // 8
                i = t % 8
                sv = sidx[j, pl.ds(i * 16, 16)]
                dv = didx[j, pl.ds(i * 16, 16)]
                val = plsc.load_gather(alive_v, [sv])
                plsc.addupdate_scatter(cnt_v, [dv], val)

        pltpu.sync_copy(cnt_v, cntp_hbm.at[wid])

    pl.run_scoped(phase_cnt, pltpu.VMEM((NPAD,), jnp.float32),
                  pltpu.VMEM((NPAD,), jnp.float32))

    def phase_rows(rows):
        @pl.loop(0, 128 * 8)
        def _(i):
            rows[0, i // 8, pl.ds((i % 8) * 16, 16)] = jnp.zeros(
                (16,), jnp.float32)

        @pl.loop(0, ROWS_PT // 128)
        def _(i):
            pltpu.sync_copy(rows.at[0],
                            agg_sh.at[pl.ds(s * ROWS_PT + i * 128, 128)])

        plsc.subcore_barrier()

        gsems = ((g00, g01, g02, g03), (g10, g11, g12, g13))
        ssems = (ssem0, ssem1)
        SUBW = CHUNK // NSUB

        def start_g(j, b):
            for u in range(NSUB):
                pltpu.async_copy(
                    h_hbm.at[sidx.at[j, pl.ds(u * SUBW, SUBW)]],
                    rows.at[b, pl.ds(u * SUBW, SUBW)], gsems[b][u])

        def wait_g(j, b):
            for u in range(NSUB):
                pltpu.make_async_copy(
                    h_hbm.at[sidx.at[j, pl.ds(u * SUBW, SUBW)]],
                    rows.at[b, pl.ds(u * SUBW, SUBW)], gsems[b][u]).wait()

        def start_s(j, b):
            pltpu.async_copy(rows.at[b], agg_sh.at[didx.at[j]], ssems[b],
                             add=True)

        def wait_s(j, b):
            pltpu.make_async_copy(rows.at[b], agg_sh.at[didx.at[j]],
                                  ssems[b]).wait()

        @pl.loop(0, NBLK)
        def _(blk):
            pltpu.sync_copy(src_hbm.at[wid, blk], sidx)
            pltpu.sync_copy(dst_hbm.at[wid, blk], didx)
            start_g(0, 0)
            wait_g(0, 0)
            start_s(0, 0)
            start_g(1, 1)

            @pl.loop(0, (BLKCH - 2) // 2)
            def _(p):
                j1 = 2 * p + 1
                wait_g(j1, 1)
                start_s(j1, 1)
                wait_s(j1 - 1, 0)
                start_g(j1 + 1, 0)
                j2 = 2 * p + 2
                wait_g(j2, 0)
                start_s(j2, 0)
                wait_s(j2 - 1, 1)
                start_g(j2 + 1, 1)

            wait_g(BLKCH - 1, 1)
            start_s(BLKCH - 1, 1)
            wait_s(BLKCH - 2, 0)
            wait_s(BLKCH - 1, 1)

    pl.run_scoped(phase_rows, pltpu.VMEM((2, CHUNK, 128), jnp.float32))

    plsc.subcore_barrier()

    pltpu.sync_copy(agg_sh.at[pl.ds(s * ROWS_PT, ROWS_PT)],
                    aggp_hbm.at[c, pl.ds(s * ROWS_PT, ROWS_PT)])


_sc_agg = functools.partial(
    pl.kernel,
    out_type=(
        jax.ShapeDtypeStruct((2, NPAD, 128), jnp.float32),
        jax.ShapeDtypeStruct((NTILES, NPAD), jnp.float32),
    ),
    mesh=_mesh,
    scratch_types=[
        pltpu.VMEM((BLKCH, CHUNK), jnp.int32),
        pltpu.VMEM((BLKCH, CHUNK), jnp.int32),
        pltpu.VMEM_SHARED((NPAD, 128), jnp.float32),
        pltpu.SemaphoreType.DMA,
        pltpu.SemaphoreType.DMA,
        pltpu.SemaphoreType.DMA,
        pltpu.SemaphoreType.DMA,
        pltpu.SemaphoreType.DMA,
        pltpu.SemaphoreType.DMA,
        pltpu.SemaphoreType.DMA,
        pltpu.SemaphoreType.DMA,
        pltpu.SemaphoreType.DMA,
        pltpu.SemaphoreType.DMA,
    ],
    compiler_params=pltpu.CompilerParams(needs_layout_passes=False),
)(_sc_agg_body)


def _tc_layer_body(n, k, h_ref, aggp_ref, cntp_ref, alive_ref,
                   wl_ref, bl_ref, wr_ref, g_ref, bt_ref, p_ref,
                   hn_ref, alive_out_ref, flat_ref):
    f32 = jnp.float32
    agg = aggp_ref[0] + aggp_ref[1]
    cnt = jnp.sum(cntp_ref[...], axis=0)
    mean = agg / jnp.maximum(cnt, 1.0)[:, None]
    h = h_ref[...]
    hc = (jnp.dot(mean, wl_ref[...], preferred_element_type=f32,
                  precision=None)
          + bl_ref[...]
          + jnp.dot(h, wr_ref[...], preferred_element_type=f32,
                    precision=None))
    alive = alive_ref[...]
    am = alive[:, None]
    mu = jnp.sum(hc * am, axis=0) / n
    dev = (hc - mu) * am
    var = jnp.sum(dev * dev, axis=0) / n
    hb = (hc - mu) / jnp.sqrt(var + 1e-5) * g_ref[...] + bt_ref[...]
    hr = jnp.maximum(hb, 0.0)
    p = p_ref[...]
    pn = jnp.sqrt(jnp.sum(p * p)) + 1e-12
    score = jnp.tanh(jnp.dot(hr, p, preferred_element_type=f32,
                             precision=None) / pn)

    bits = lax.bitcast_convert_type(score, jnp.int32)
    key = jnp.where(bits >= 0, bits, bits ^ jnp.int32(0x7FFFFFFF))
    ukey = lax.bitcast_convert_type(key ^ jnp.int32(-2147483648), jnp.uint32)
    ukey = jnp.where(alive > 0.0, ukey, jnp.uint32(0))

    def _thr(_, carry):
        lo, hi = carry
        span = hi - lo
        mid = lo + (span >> jnp.uint32(1)) + (span & jnp.uint32(1))
        ge = jnp.sum((ukey >= mid).astype(jnp.int32))
        ok = ge >= k
        return (jnp.where(ok, mid, lo), jnp.where(ok, hi, mid - jnp.uint32(1)))

    t, _ = lax.fori_loop(0, 32, _thr,
                         (jnp.uint32(0), jnp.uint32(0xFFFFFFFF)))

    above = ukey > t
    ties = ukey == t
    need = k - jnp.sum(above.astype(jnp.int32))
    idx = lax.broadcasted_iota(jnp.int32, (NPAD,), 0)

    def _cut(_, carry):
        lo, hi = carry
        mid = (lo + hi) // 2
        q = jnp.sum((ties & (idx < mid)).astype(jnp.int32)) >= need
        return (jnp.where(q, lo, mid), jnp.where(q, mid, hi))

    _, m = lax.fori_loop(0, 14, _cut, (jnp.int32(0), jnp.int32(NPAD)))

    keep = above | (ties & (idx < m))
    keep_f = keep.astype(f32)
    hn = hr * (score * keep_f)[:, None]
    hn_ref[...] = hn
    alive_out_ref[...] = keep_f
    add_p = jnp.sum(hn, axis=0)
    neg = jnp.float32(-3.4028235e38)
    max_p = jnp.max(jnp.where(keep_f[:, None] > 0.0, hn, neg), axis=0)
    flat_ref[...] = jnp.concatenate([add_p, max_p]).reshape(1, 256)


def _tc_layer(n, k, h, aggp, cntp, alive, wl, bl, wr, g, bt, p):
    return pl.pallas_call(
        functools.partial(_tc_layer_body, n, k),
        out_shape=(
            jax.ShapeDtypeStruct((NPAD, 128), jnp.float32),
            jax.ShapeDtypeStruct((NPAD,), jnp.float32),
            jax.ShapeDtypeStruct((1, 256), jnp.float32),
        ),
        compiler_params=pltpu.CompilerParams(
            vmem_limit_bytes=100 * 1024 * 1024),
    )(h, aggp, cntp, alive, wl, bl, wr, g, bt, p)


def _tc_head_body(f1, f2, f3, f4, w5_ref, b5_ref, w6_ref, b6_ref, out_ref):
    f32 = jnp.float32
    flat = jnp.concatenate([f1[...], f2[...], f3[...], f4[...]], axis=-1)
    hid = jnp.maximum(
        jnp.dot(flat, w5_ref[...], preferred_element_type=f32,
                precision=None) + b5_ref[...], 0.0)
    out_ref[...] = (jnp.dot(hid, w6_ref[...], preferred_element_type=f32,
                            precision=None) + b6_ref[...])


def kernel(x, edge_index, batch, Wl1, bl1, Wr1, g1, bt1, p1, Wl2, bl2, Wr2,
           g2, bt2, p2, Wl3, bl3, Wr3, g3, bt3, p3, Wl4, bl4, Wr4, g4, bt4,
           p4, W5, b5, W6, b6):
    src = edge_index[0]
    dst = edge_index[1]
    h = jnp.zeros((NPAD, 128), jnp.float32).at[:N, :D].set(x)
    pad_e = jnp.full((EPAD - E,), N, jnp.int32)
    src3 = jnp.concatenate([src, pad_e]).reshape(NTILES, NBLK, BLKCH, CHUNK)
    dst3 = jnp.concatenate([dst, pad_e]).reshape(NTILES, NBLK, BLKCH, CHUNK)
    alive = (jnp.arange(NPAD) < N).astype(jnp.float32)

    params = [(Wl1, bl1, Wr1, g1, bt1, p1), (Wl2, bl2, Wr2, g2, bt2, p2),
              (Wl3, bl3, Wr3, g3, bt3, p3), (Wl4, bl4, Wr4, g4, bt4, p4)]
    n = N
    flats = []
    for (wl, bl, wr, g, bt, p) in params:
        k = int(np.ceil(0.8 * n))
        aggp, cntp = _sc_agg(h, src3, dst3, alive)
        h, alive, flat = _tc_layer(n, k, h, aggp, cntp, alive,
                                   wl, bl, wr, g, bt, p)
        flats.append(flat)
        n = k

    return pl.pallas_call(
        _tc_head_body,
        out_shape=jax.ShapeDtypeStruct((1, NC_OUT), jnp.float32),
    )(flats[0], flats[1], flats[2], flats[3], W5, b5, W6, b6)


# speedup vs baseline: 1.0413x; 1.0413x over previous
import functools

import jax
import jax.numpy as jnp
import numpy as np
from jax import lax
from jax.experimental import pallas as pl
from jax.experimental.pallas import tpu as pltpu
from jax.experimental.pallas import tpu_sc as plsc

N = 10000
E = 320000
D = 128
H = 128
NC_OUT = 10

NPAD = 10240
NTILES = 32
CHUNK = 128
NSUB = 4
BLKCH = 16
NBLK = 5
NCH = NBLK * BLKCH
EPT = NCH * CHUNK
EPAD = EPT * NTILES
ROWS_PT = NPAD // 16

_mesh = plsc.VectorSubcoreMesh(core_axis_name="c", subcore_axis_name="s")


def _sc_agg_body(h_hbm, src_hbm, dst_hbm, alive_hbm, aggp_hbm, cntp_hbm,
                 sidx, didx, agg_sh,
                 g00, g01, g02, g03, g10, g11, g12, g13, ssem0, ssem1):
    c = lax.axis_index("c")
    s = lax.axis_index("s")
    wid = c * 16 + s

    def phase_cnt(alive_v, cnt_v):
        pltpu.sync_copy(alive_hbm, alive_v)

        @pl.loop(0, NPAD // 16)
        def _(i):
            cnt_v[pl.ds(i * 16, 16)] = jnp.zeros((16,), jnp.float32)

        @pl.loop(0, NBLK)
        def _(b):
            pltpu.sync_copy(src_hbm.at[wid, b], sidx)
            pltpu.sync_copy(dst_hbm.at[wid, b], didx)

            @pl.loop(0, BLKCH)
            def _(j):
                for i in range(8):
                    sv = sidx[j, pl.ds(i * 16, 16)]
                    dv = didx[j, pl.ds(i * 16, 16)]
                    val = plsc.load_gather(alive_v, [sv])
                    plsc.addupdate_scatter(cnt_v, [dv], val)

        pltpu.sync_copy(cnt_v, cntp_hbm.at[wid])

    pl.run_scoped(phase_cnt, pltpu.VMEM((NPAD,), jnp.float32),
                  pltpu.VMEM((NPAD,), jnp.float32))

    def phase_rows(rows):
        @pl.loop(0, CHUNK)
        def _(r):
            for u in range(8):
                rows[0, r, pl.ds(u * 16, 16)] = jnp.zeros((16,), jnp.float32)

        @pl.loop(0, ROWS_PT // CHUNK)
        def _(i):
            pltpu.sync_copy(rows.at[0],
                            agg_sh.at[pl.ds(s * ROWS_PT + i * CHUNK, CHUNK)])

        plsc.subcore_barrier()

        gsems = ((g00, g01, g02, g03), (g10, g11, g12, g13))
        ssems = (ssem0, ssem1)
        SUBW = CHUNK // NSUB

        def start_g(j, b):
            for u in range(NSUB):
                pltpu.async_copy(
                    h_hbm.at[sidx.at[j, pl.ds(u * SUBW, SUBW)]],
                    rows.at[b, pl.ds(u * SUBW, SUBW)], gsems[b][u])

        def wait_g(j, b):
            for u in range(NSUB):
                pltpu.make_async_copy(
                    h_hbm.at[sidx.at[j, pl.ds(u * SUBW, SUBW)]],
                    rows.at[b, pl.ds(u * SUBW, SUBW)], gsems[b][u]).wait()

        def start_s(j, b):
            pltpu.async_copy(rows.at[b], agg_sh.at[didx.at[j]], ssems[b],
                             add=True)

        def wait_s(j, b):
            pltpu.make_async_copy(rows.at[b], agg_sh.at[didx.at[j]],
                                  ssems[b]).wait()

        @pl.loop(0, NBLK)
        def _(blk):
            pltpu.sync_copy(src_hbm.at[wid, blk], sidx)
            pltpu.sync_copy(dst_hbm.at[wid, blk], didx)
            start_g(0, 0)
            start_g(1, 1)
            wait_g(0, 0)
            start_s(0, 0)

            @pl.loop(0, (BLKCH - 2) // 2)
            def _(p):
                j1 = 2 * p + 1
                wait_s(j1 - 1, 0)
                start_g(j1 + 1, 0)
                wait_g(j1, 1)
                start_s(j1, 1)
                j2 = 2 * p + 2
                wait_s(j2 - 1, 1)
                start_g(j2 + 1, 1)
                wait_g(j2, 0)
                start_s(j2, 0)

            wait_s(BLKCH - 2, 0)
            wait_g(BLKCH - 1, 1)
            start_s(BLKCH - 1, 1)
            wait_s(BLKCH - 1, 1)

    pl.run_scoped(phase_rows, pltpu.VMEM((2, CHUNK, 128), jnp.float32))

    plsc.subcore_barrier()

    pltpu.sync_copy(agg_sh.at[pl.ds(s * ROWS_PT, ROWS_PT)],
                    aggp_hbm.at[c, pl.ds(s * ROWS_PT, ROWS_PT)])


_sc_agg = functools.partial(
    pl.kernel,
    out_type=(
        jax.ShapeDtypeStruct((2, NPAD, 128), jnp.float32),
        jax.ShapeDtypeStruct((NTILES, NPAD), jnp.float32),
    ),
    mesh=_mesh,
    scratch_types=[
        pltpu.VMEM((BLKCH, CHUNK), jnp.int32),
        pltpu.VMEM((BLKCH, CHUNK), jnp.int32),
        pltpu.VMEM_SHARED((NPAD, 128), jnp.float32),
        pltpu.SemaphoreType.DMA,
        pltpu.SemaphoreType.DMA,
        pltpu.SemaphoreType.DMA,
        pltpu.SemaphoreType.DMA,
        pltpu.SemaphoreType.DMA,
        pltpu.SemaphoreType.DMA,
        pltpu.SemaphoreType.DMA,
        pltpu.SemaphoreType.DMA,
        pltpu.SemaphoreType.DMA,
        pltpu.SemaphoreType.DMA,
    ],
    compiler_params=pltpu.CompilerParams(needs_layout_passes=False),
)(_sc_agg_body)


def _tc_layer_body(n, k, h_ref, aggp_ref, cntp_ref, alive_ref,
                   wl_ref, bl_ref, wr_ref, g_ref, bt_ref, p_ref,
                   hn_ref, alive_out_ref, flat_ref):
    f32 = jnp.float32
    agg = aggp_ref[0] + aggp_ref[1]
    cnt = jnp.sum(cntp_ref[...], axis=0)
    mean = agg / jnp.maximum(cnt, 1.0)[:, None]
    h = h_ref[...]
    hc = (jnp.dot(mean, wl_ref[...], preferred_element_type=f32)
          + bl_ref[...]
          + jnp.dot(h, wr_ref[...], preferred_element_type=f32))
    alive = alive_ref[...]
    am = alive[:, None]
    mu = jnp.sum(hc * am, axis=0) / n
    dev = (hc - mu) * am
    var = jnp.sum(dev * dev, axis=0) / n
    hb = (hc - mu) / jnp.sqrt(var + 1e-5) * g_ref[...] + bt_ref[...]
    hr = jnp.maximum(hb, 0.0)
    p = p_ref[...]
    pn = jnp.sqrt(jnp.sum(p * p)) + 1e-12
    score = jnp.tanh(jnp.dot(hr, p, preferred_element_type=f32) / pn)

    bits = lax.bitcast_convert_type(score, jnp.int32)
    key = jnp.where(bits >= 0, bits, bits ^ jnp.int32(0x7FFFFFFF))
    ukey = lax.bitcast_convert_type(key ^ jnp.int32(-2147483648), jnp.uint32)
    ukey = jnp.where(alive > 0.0, ukey, jnp.uint32(0))

    def _thr(_, carry):
        lo, hi = carry
        span = hi - lo
        mid = lo + (span >> jnp.uint32(1)) + (span & jnp.uint32(1))
        ge = jnp.sum((ukey >= mid).astype(jnp.int32))
        ok = ge >= k
        return (jnp.where(ok, mid, lo), jnp.where(ok, hi, mid - jnp.uint32(1)))

    t, _ = lax.fori_loop(0, 32, _thr,
                         (jnp.uint32(0), jnp.uint32(0xFFFFFFFF)))

    above = ukey > t
    ties = ukey == t
    need = k - jnp.sum(above.astype(jnp.int32))
    idx = lax.broadcasted_iota(jnp.int32, (NPAD,), 0)

    def _cut(_, carry):
        lo, hi = carry
        mid = (lo + hi) // 2
        q = jnp.sum((ties & (idx < mid)).astype(jnp.int32)) >= need
        return (jnp.where(q, lo, mid), jnp.where(q, mid, hi))

    _, m = lax.fori_loop(0, 14, _cut, (jnp.int32(0), jnp.int32(NPAD)))

    keep = above | (ties & (idx < m))
    keep_f = keep.astype(f32)
    hn = hr * (score * keep_f)[:, None]
    hn_ref[...] = hn
    alive_out_ref[...] = keep_f
    add_p = jnp.sum(hn, axis=0)
    neg = jnp.float32(-3.4028235e38)
    max_p = jnp.max(jnp.where(keep_f[:, None] > 0.0, hn, neg), axis=0)
    flat_ref[...] = jnp.concatenate([add_p, max_p]).reshape(1, 256)


def _tc_layer(n, k, h, aggp, cntp, alive, wl, bl, wr, g, bt, p):
    return pl.pallas_call(
        functools.partial(_tc_layer_body, n, k),
        out_shape=(
            jax.ShapeDtypeStruct((NPAD, 128), jnp.float32),
            jax.ShapeDtypeStruct((NPAD,), jnp.float32),
            jax.ShapeDtypeStruct((1, 256), jnp.float32),
        ),
        compiler_params=pltpu.CompilerParams(
            vmem_limit_bytes=100 * 1024 * 1024),
    )(h, aggp, cntp, alive, wl, bl, wr, g, bt, p)


def _tc_head_body(f1, f2, f3, f4, w5_ref, b5_ref, w6_ref, b6_ref, out_ref):
    f32 = jnp.float32
    flat = jnp.concatenate([f1[...], f2[...], f3[...], f4[...]], axis=-1)
    hid = jnp.maximum(
        jnp.dot(flat, w5_ref[...], preferred_element_type=f32) + b5_ref[...],
        0.0)
    out_ref[...] = (jnp.dot(hid, w6_ref[...], preferred_element_type=f32)
                    + b6_ref[...])


def kernel(x, edge_index, batch, Wl1, bl1, Wr1, g1, bt1, p1, Wl2, bl2, Wr2,
           g2, bt2, p2, Wl3, bl3, Wr3, g3, bt3, p3, Wl4, bl4, Wr4, g4, bt4,
           p4, W5, b5, W6, b6):
    src = edge_index[0]
    dst = edge_index[1]
    h = jnp.zeros((NPAD, 128), jnp.float32).at[:N, :D].set(x)
    pad_e = jnp.full((EPAD - E,), N, jnp.int32)
    src3 = jnp.concatenate([src, pad_e]).reshape(NTILES, NBLK, BLKCH, CHUNK)
    dst3 = jnp.concatenate([dst, pad_e]).reshape(NTILES, NBLK, BLKCH, CHUNK)
    alive = (jnp.arange(NPAD) < N).astype(jnp.float32)

    params = [(Wl1, bl1, Wr1, g1, bt1, p1), (Wl2, bl2, Wr2, g2, bt2, p2),
              (Wl3, bl3, Wr3, g3, bt3, p3), (Wl4, bl4, Wr4, g4, bt4, p4)]
    n = N
    flats = []
    for (wl, bl, wr, g, bt, p) in params:
        k = int(np.ceil(0.8 * n))
        aggp, cntp = _sc_agg(h, src3, dst3, alive)
        h, alive, flat = _tc_layer(n, k, h, aggp, cntp, alive,
                                   wl, bl, wr, g, bt, p)
        flats.append(flat)
        n = k

    return pl.pallas_call(
        _tc_head_body,
        out_shape=jax.ShapeDtypeStruct((1, NC_OUT), jnp.float32),
    )(flats[0], flats[1], flats[2], flats[3], W5, b5, W6, b6)
